# baseline (speedup 1.0000x reference)
import jax
import jax.numpy as jnp
from jax import lax
from jax.experimental import pallas as pl
from jax.experimental.pallas import tpu as pltpu
from jax.experimental.pallas import tpu_sc as plsc

B = 4096
H = 50
D = 32
L = 16
NC = 2
NS = 16
NW = NC * NS
BPW = B // NW
INV_H = 1.0 / H


def _body(ids_hbm, table_hbm, out_hbm, idx_v, buf0, buf1, out_v, sem0, sem1):
    wid = lax.axis_index("s") * NC + lax.axis_index("c")
    base = wid * BPW

    pltpu.sync_copy(ids_hbm.at[pl.ds(base, BPW)], idx_v)

    def start(j, buf, sem):
        pltpu.async_copy(table_hbm.at[idx_v.at[j]], buf, sem)

    def wait(j, buf, sem):
        pltpu.make_async_copy(table_hbm.at[idx_v.at[j]], buf, sem).wait()

    def reduce_row(j, buf):
        def add_row(t, carry):
            a0, a1 = carry
            return (a0 + buf[t, pl.ds(0, L)], a1 + buf[t, pl.ds(L, L)])

        z = jnp.zeros((L,), jnp.float32)
        a0, a1 = lax.fori_loop(0, H, add_row, (z, z), unroll=5)
        out_v[j, pl.ds(0, L)] = a0 * INV_H
        out_v[j, pl.ds(L, L)] = a1 * INV_H

    start(0, buf0, sem0)
    start(1, buf1, sem1)

    def ring(i, _):
        j0 = 2 * i
        wait(j0, buf0, sem0)
        reduce_row(j0, buf0)
        start(j0 + 2, buf0, sem0)
        wait(j0 + 1, buf1, sem1)
        reduce_row(j0 + 1, buf1)
        start(j0 + 3, buf1, sem1)
        return 0

    lax.fori_loop(0, BPW // 2 - 1, ring, 0)

    wait(BPW - 2, buf0, sem0)
    reduce_row(BPW - 2, buf0)
    wait(BPW - 1, buf1, sem1)
    reduce_row(BPW - 1, buf1)

    pltpu.sync_copy(out_v, out_hbm.at[pl.ds(base, BPW)])


def kernel(input_ids, pretrained_embeddings):
    mesh = plsc.VectorSubcoreMesh(
        core_axis_name="c", subcore_axis_name="s",
        num_cores=NC, num_subcores=NS,
    )
    run = pl.kernel(
        _body,
        out_type=jax.ShapeDtypeStruct((B, D), jnp.float32),
        mesh=mesh,
        compiler_params=pltpu.CompilerParams(use_tc_tiling_on_sc=False),
        scratch_types=[
            pltpu.VMEM((BPW, H), jnp.int32),
            pltpu.VMEM((H, D), jnp.float32),
            pltpu.VMEM((H, D), jnp.float32),
            pltpu.VMEM((BPW, D), jnp.float32),
            pltpu.SemaphoreType.DMA,
            pltpu.SemaphoreType.DMA,
        ],
    )
    return run(input_ids, pretrained_embeddings)


# speedup vs baseline: 1.0569x; 1.0569x over previous
import jax
import jax.numpy as jnp
from jax import lax
from jax.experimental import pallas as pl
from jax.experimental.pallas import tpu as pltpu
from jax.experimental.pallas import tpu_sc as plsc

B = 4096
H = 50
D = 32
L = 16
NC = 2
NS = 16
NW = NC * NS
BPW = B // NW
G = 2
RPW = BPW // G
NBUF = 4
INV_H = 1.0 / H


def _body(ids_hbm, table_hbm, out_hbm, idx_v, bufs, out_v, sems):
    wid = lax.axis_index("s") * NC + lax.axis_index("c")
    base = wid * RPW

    pltpu.sync_copy(ids_hbm.at[pl.ds(base, RPW)], idx_v)

    def start(c, b):
        pltpu.async_copy(table_hbm.at[idx_v.at[c]], bufs[b], sems[b])

    def wait(c, b):
        pltpu.make_async_copy(table_hbm.at[idx_v.at[c]], bufs[b], sems[b]).wait()

    def reduce_row(c, b):
        buf = bufs[b]
        for h in range(G):
            def add_row(t, carry):
                a0, a1 = carry
                return (a0 + buf[h * H + t, pl.ds(0, L)],
                        a1 + buf[h * H + t, pl.ds(L, L)])

            z = jnp.zeros((L,), jnp.float32)
            a0, a1 = lax.fori_loop(0, H, add_row, (z, z), unroll=5)
            out_v[G * c + h, pl.ds(0, L)] = a0 * INV_H
            out_v[G * c + h, pl.ds(L, L)] = a1 * INV_H

    for b in range(NBUF):
        start(b, b)

    def ring(i, _):
        c0 = NBUF * i
        for b in range(NBUF):
            wait(c0 + b, b)
            reduce_row(c0 + b, b)
            start(c0 + b + NBUF, b)
        return 0

    lax.fori_loop(0, RPW // NBUF - 1, ring, 0)

    for b in range(NBUF):
        c = RPW - NBUF + b
        wait(c, b)
        reduce_row(c, b)

    pltpu.sync_copy(out_v, out_hbm.at[pl.ds(wid * BPW, BPW)])


def kernel(input_ids, pretrained_embeddings):
    ids2 = input_ids.reshape(B // G, G * H)
    mesh = plsc.VectorSubcoreMesh(
        core_axis_name="c", subcore_axis_name="s",
        num_cores=NC, num_subcores=NS,
    )
    run = pl.kernel(
        _body,
        out_type=jax.ShapeDtypeStruct((B, D), jnp.float32),
        mesh=mesh,
        compiler_params=pltpu.CompilerParams(use_tc_tiling_on_sc=False),
        scratch_types=[
            pltpu.VMEM((RPW, G * H), jnp.int32),
            [pltpu.VMEM((G * H, D), jnp.float32) for _ in range(NBUF)],
            pltpu.VMEM((BPW, D), jnp.float32),
            [pltpu.SemaphoreType.DMA for _ in range(NBUF)],
        ],
    )
    return run(ids2, pretrained_embeddings)
